# baseline (speedup 1.0000x reference)
import functools

import jax
import jax.numpy as jnp
from jax import lax
from jax.experimental import pallas as pl
from jax.experimental.pallas import tpu as pltpu
from jax.experimental.pallas import tpu_sc as plsc

EMBED = 32
NC = 2
NS = 16
NW = NC * NS
LANES = 16

BATCH = 16384
SEQ = 50
NBT = BATCH // 128
BT_PER_W = NBT // NW
ROWS = BT_PER_W * 128
N_PAIR = SEQ // 2


def _make_gather():
    mesh = plsc.VectorSubcoreMesh(core_axis_name="c", subcore_axis_name="s")

    @functools.partial(
        pl.kernel,
        mesh=mesh,
        out_type=jax.ShapeDtypeStruct((SEQ, EMBED // 8, NBT, 1024),
                                      jnp.float32),
        scratch_types=[
            pltpu.VMEM((ROWS,), jnp.int32),
            pltpu.VMEM((ROWS,), jnp.int32),
            pltpu.VMEM((ROWS, EMBED), jnp.float32),
            pltpu.VMEM((ROWS, EMBED), jnp.float32),
            pltpu.VMEM((BT_PER_W * 4 * 1024,), jnp.float32),
            pltpu.VMEM((BT_PER_W * 4 * 1024,), jnp.float32),
            pltpu.SemaphoreType.DMA,
            pltpu.SemaphoreType.DMA,
            pltpu.SemaphoreType.DMA,
            pltpu.SemaphoreType.DMA,
        ],
        compiler_params=pltpu.CompilerParams(
            use_tc_tiling_on_sc=False, needs_layout_passes=False),
    )
    def k(ids_hbm, table_hbm, out_hbm, idx0, idx1, rows0, rows1, tb0, tb1,
          gs0, gs1, ss0, ss1):
        wid = lax.axis_index("s") * NC + lax.axis_index("c")
        bt0 = wid * BT_PER_W
        idx = (idx0, idx1)
        rows = (rows0, rows1)
        tbuf = (tb0, tb1)
        gs = (gs0, gs1)
        ss = (ss0, ss1)
        lane = lax.iota(jnp.int32, LANES)

        rvecs = [lane + (q * 128 + lg * 16)
                 for q in range(BT_PER_W) for lg in range(8)]

        def transpose_rows(p):
            @plsc.parallel_loop(0, EMBED, 1, unroll=1)
            def ebody(e):
                col = jnp.full((LANES,), e, jnp.int32)
                off_e = (e >> 3) * 1024 + (e & 7) * 128
                depth = 4

                def st(m, val):
                    q, lg = divmod(m, 8)
                    tbuf[p][pl.ds(off_e + q * 4096 + lg * 16, 16)] = val

                pending = []
                for m, rv in enumerate(rvecs):
                    pending.append((m, plsc.load_gather(rows[p], [rv, col])))
                    if len(pending) > depth:
                        st(*pending.pop(0))
                for mv in pending:
                    st(*mv)

        def fire_stores(p, l):
            for q in range(BT_PER_W):
                for eT in range(EMBED // 8):
                    pltpu.async_copy(
                        tbuf[p].at[pl.ds(q * 4096 + eT * 1024, 1024)],
                        out_hbm.at[l, eT, bt0 + q],
                        ss[p])

        def drain_stores(p):
            for _ in range(BT_PER_W * (EMBED // 8)):
                pltpu.make_async_copy(
                    tbuf[p].at[pl.ds(0, 1024)], out_hbm.at[0, 0, 0],
                    ss[p]).wait()

        def body(g, carry):
            for b in (0, 1):
                l = 2 * g + b
                p = 1 - b

                @pl.when(g > 0)
                def _wait_prev_stores():
                    drain_stores(b)

                pltpu.sync_copy(
                    ids_hbm.at[pl.ds(l * BATCH + bt0 * 128, ROWS)], idx[b])
                pltpu.async_copy(table_hbm.at[idx[b]], rows[b], gs[b])

                def _retire():
                    pltpu.make_async_copy(
                        table_hbm.at[idx[p]], rows[p], gs[p]).wait()
                    transpose_rows(p)
                    fire_stores(p, l - 1)

                if b == 0:
                    pl.when(g > 0)(_retire)
                else:
                    _retire()
            return carry

        lax.fori_loop(0, N_PAIR, body, 0)

        pltpu.make_async_copy(table_hbm.at[idx1], rows1, gs1).wait()
        transpose_rows(1)
        fire_stores(1, SEQ - 1)
        drain_stores(0)
        drain_stores(1)

    return k


_gather = _make_gather()


def kernel(input_ids, table):
    flat_ids_t = input_ids.T.reshape(SEQ * BATCH)
    out5 = _gather(flat_ids_t, table)
    out = out5.reshape(SEQ, EMBED // 8, NBT, 8, 128)
    out = out.transpose(2, 4, 0, 1, 3)
    return out.reshape(BATCH, SEQ, EMBED)


# speedup vs baseline: 1.1092x; 1.1092x over previous
import functools

import jax
import jax.numpy as jnp
from jax import lax
from jax.experimental import pallas as pl
from jax.experimental.pallas import tpu as pltpu
from jax.experimental.pallas import tpu_sc as plsc

EMBED = 32
NC = 2
NS = 16
NW = NC * NS
LANES = 16

BATCH = 16384
SEQ = 50
NBT = BATCH // 128
BT_PER_W = NBT // NW
ROWS = BT_PER_W * 128
N_PAIR = SEQ // 2


def _make_gather():
    mesh = plsc.VectorSubcoreMesh(core_axis_name="c", subcore_axis_name="s")

    @functools.partial(
        pl.kernel,
        mesh=mesh,
        out_type=jax.ShapeDtypeStruct((SEQ, EMBED // 8, NBT, 1024),
                                      jnp.float32),
        scratch_types=[
            pltpu.VMEM((ROWS,), jnp.int32),
            pltpu.VMEM((ROWS,), jnp.int32),
            pltpu.VMEM((ROWS, EMBED), jnp.float32),
            pltpu.VMEM((ROWS, EMBED), jnp.float32),
            pltpu.VMEM((BT_PER_W * 4 * 1024,), jnp.float32),
            pltpu.VMEM((BT_PER_W * 4 * 1024,), jnp.float32),
            pltpu.SemaphoreType.DMA,
            pltpu.SemaphoreType.DMA,
            pltpu.SemaphoreType.DMA,
            pltpu.SemaphoreType.DMA,
        ],
        compiler_params=pltpu.CompilerParams(
            use_tc_tiling_on_sc=False, needs_layout_passes=False),
    )
    def k(ids_hbm, table_hbm, out_hbm, idx0, idx1, rows0, rows1, tb0, tb1,
          gs0, gs1, ss0, ss1):
        wid = lax.axis_index("s") * NC + lax.axis_index("c")
        bt0 = wid * BT_PER_W
        idx = (idx0, idx1)
        rows = (rows0, rows1)
        tbuf = (tb0, tb1)
        gs = (gs0, gs1)
        ss = (ss0, ss1)
        lane = lax.iota(jnp.int32, LANES)

        rvecs = [lane + (q * 128 + lg * 16)
                 for q in range(BT_PER_W) for lg in range(8)]

        def transpose_rows(p):
            @plsc.parallel_loop(0, EMBED, 1, unroll=1)
            def ebody(e):
                col = jnp.full((LANES,), e, jnp.int32)
                off_e = (e >> 3) * 1024 + (e & 7) * 128
                for m, rv in enumerate(rvecs):
                    q, lg = divmod(m, 8)
                    val = plsc.load_gather(rows[p], [rv, col])
                    tbuf[p][pl.ds(off_e + q * 4096 + lg * 16, 16)] = val

        def fire_stores(p, l):
            for q in range(BT_PER_W):
                for eT in range(EMBED // 8):
                    pltpu.async_copy(
                        tbuf[p].at[pl.ds(q * 4096 + eT * 1024, 1024)],
                        out_hbm.at[l, eT, bt0 + q],
                        ss[p])

        def drain_stores(p):
            for _ in range(BT_PER_W * (EMBED // 8)):
                pltpu.make_async_copy(
                    tbuf[p].at[pl.ds(0, 1024)], out_hbm.at[0, 0, 0],
                    ss[p]).wait()

        def body(g, carry):
            for b in (0, 1):
                l = 2 * g + b
                p = 1 - b

                @pl.when(g > 0)
                def _wait_prev_stores():
                    drain_stores(b)

                pltpu.sync_copy(
                    ids_hbm.at[pl.ds(l * BATCH + bt0 * 128, ROWS)], idx[b])
                pltpu.async_copy(table_hbm.at[idx[b]], rows[b], gs[b])

                def _retire():
                    pltpu.make_async_copy(
                        table_hbm.at[idx[p]], rows[p], gs[p]).wait()
                    transpose_rows(p)
                    fire_stores(p, l - 1)

                if b == 0:
                    pl.when(g > 0)(_retire)
                else:
                    _retire()
            return carry

        lax.fori_loop(0, N_PAIR, body, 0)

        pltpu.make_async_copy(table_hbm.at[idx1], rows1, gs1).wait()
        transpose_rows(1)
        fire_stores(1, SEQ - 1)
        drain_stores(0)
        drain_stores(1)

    return k


_gather = _make_gather()


def kernel(input_ids, table):
    flat_ids_t = input_ids.T.reshape(SEQ * BATCH)
    out5 = _gather(flat_ids_t, table)
    out = out5.reshape(SEQ, EMBED // 8, NBT, 8, 128)
    out = out.transpose(2, 4, 0, 1, 3)
    return out.reshape(BATCH, SEQ, EMBED)


# speedup vs baseline: 1.5337x; 1.3827x over previous
import functools

import jax
import jax.numpy as jnp
from jax import lax
from jax.experimental import pallas as pl
from jax.experimental.pallas import tpu as pltpu
from jax.experimental.pallas import tpu_sc as plsc

EMBED = 32
NC = 2
NS = 16
NW = NC * NS
LANES = 16

BATCH = 16384
SEQ = 50
NBT = BATCH // 128
BT_PER_W = NBT // NW
ROWS = BT_PER_W * 128
N_PAIR = SEQ // 2


def _make_gather():
    mesh = plsc.VectorSubcoreMesh(core_axis_name="c", subcore_axis_name="s")

    @functools.partial(
        pl.kernel,
        mesh=mesh,
        out_type=jax.ShapeDtypeStruct((SEQ, EMBED // 8, NBT, 1024),
                                      jnp.float32),
        scratch_types=[
            pltpu.VMEM((ROWS,), jnp.int32),
            pltpu.VMEM((ROWS,), jnp.int32),
            pltpu.VMEM((ROWS, EMBED), jnp.float32),
            pltpu.VMEM((ROWS, EMBED), jnp.float32),
            pltpu.VMEM((BT_PER_W * 4 * 1024,), jnp.float32),
            pltpu.VMEM((BT_PER_W * 4 * 1024,), jnp.float32),
            pltpu.SemaphoreType.DMA,
            pltpu.SemaphoreType.DMA,
            pltpu.SemaphoreType.DMA,
            pltpu.SemaphoreType.DMA,
        ],
        compiler_params=pltpu.CompilerParams(
            use_tc_tiling_on_sc=False, needs_layout_passes=False),
    )
    def k(ids_hbm, table_hbm, out_hbm, idx0, idx1, rows0, rows1, tb0, tb1,
          gs0, gs1, ss0, ss1):
        wid = lax.axis_index("s") * NC + lax.axis_index("c")
        bt0 = wid * BT_PER_W
        idx = (idx0, idx1)
        rows = (rows0, rows1)
        tbuf = (tb0, tb1)
        gs = (gs0, gs1)
        ss = (ss0, ss1)
        lane = lax.iota(jnp.int32, LANES)

        gdn = lax.GatherDimensionNumbers(
            offset_dims=(), collapsed_slice_dims=(0,), start_index_map=(0,))
        perms = {s: (lane ^ s)[:, None] for s in (1, 2, 4, 8)}
        masks = {s: (lane & s) == 0 for s in (1, 2, 4, 8)}

        def _shuf(v, s):
            return lax.gather(v, perms[s], gdn, slice_sizes=(1,),
                              mode=lax.GatherScatterMode.PROMISE_IN_BOUNDS)

        def transpose_rows(p):
            @plsc.parallel_loop(0, 64, 1, unroll=1)
            def mbody(m):
                q = m >> 4
                lg = (m >> 1) & 7
                eh = m & 1
                r0 = q * 128 + lg * 16
                regs = [rows[p][r0 + i, pl.ds(eh * 16, LANES)]
                        for i in range(LANES)]
                for s in (1, 2, 4, 8):
                    msk = masks[s]
                    nxt = list(regs)
                    for i in range(LANES):
                        if i & s == 0:
                            j = i | s
                            a, b2 = regs[i], regs[j]
                            nxt[i] = jnp.where(msk, a, _shuf(b2, s))
                            nxt[j] = jnp.where(msk, _shuf(a, s), b2)
                    regs = nxt
                base = q * 4096 + lg * 16
                for k in range(LANES):
                    e = eh * 16 + k
                    off = (e >> 3) * 1024 + (e & 7) * 128 + base
                    tbuf[p][pl.ds(off, LANES)] = regs[k]

        def fire_stores(p, l):
            for q in range(BT_PER_W):
                for eT in range(EMBED // 8):
                    pltpu.async_copy(
                        tbuf[p].at[pl.ds(q * 4096 + eT * 1024, 1024)],
                        out_hbm.at[l, eT, bt0 + q],
                        ss[p])

        def drain_stores(p):
            for _ in range(BT_PER_W * (EMBED // 8)):
                pltpu.make_async_copy(
                    tbuf[p].at[pl.ds(0, 1024)], out_hbm.at[0, 0, 0],
                    ss[p]).wait()

        def body(g, carry):
            for b in (0, 1):
                l = 2 * g + b
                p = 1 - b

                @pl.when(g > 0)
                def _wait_prev_stores():
                    drain_stores(b)

                pltpu.sync_copy(
                    ids_hbm.at[pl.ds(l * BATCH + bt0 * 128, ROWS)], idx[b])
                pltpu.async_copy(table_hbm.at[idx[b]], rows[b], gs[b])

                def _retire():
                    pltpu.make_async_copy(
                        table_hbm.at[idx[p]], rows[p], gs[p]).wait()
                    transpose_rows(p)
                    fire_stores(p, l - 1)

                if b == 0:
                    pl.when(g > 0)(_retire)
                else:
                    _retire()
            return carry

        lax.fori_loop(0, N_PAIR, body, 0)

        pltpu.make_async_copy(table_hbm.at[idx1], rows1, gs1).wait()
        transpose_rows(1)
        fire_stores(1, SEQ - 1)
        drain_stores(0)
        drain_stores(1)

    return k


_gather = _make_gather()


def kernel(input_ids, table):
    flat_ids_t = input_ids.T.reshape(SEQ * BATCH)
    out5 = _gather(flat_ids_t, table)
    out = out5.reshape(SEQ, EMBED // 8, NBT, 8, 128)
    out = out.transpose(2, 4, 0, 1, 3)
    return out.reshape(BATCH, SEQ, EMBED)
